# baseline (speedup 1.0000x reference)
import functools

import jax
import jax.numpy as jnp
from jax import lax
from jax.experimental import pallas as pl
from jax.experimental.pallas import tpu as pltpu
from jax.experimental.pallas import tpu_sc as plsc

B = 16384
SEQ = 200
D = 64
NC, NS, L = 2, 16, 16
NW = NC * NS
BPW = B // NW


@functools.partial(
    pl.kernel,
    out_type=jax.ShapeDtypeStruct((8, 128, 8, 128), jnp.float32),
    mesh=plsc.VectorSubcoreMesh(
        core_axis_name="c", subcore_axis_name="s",
        num_cores=NC, num_subcores=NS),
    scratch_types=[
        pltpu.VMEM((BPW,), jnp.int32),
        pltpu.VMEM((BPW,), jnp.int32),
        pltpu.VMEM((4, 128), jnp.int32),
        pltpu.VMEM((BPW,), jnp.int32),
        pltpu.VMEM((BPW, 2 * D), jnp.float32),
        pltpu.VMEM((8, 4, 8, 128), jnp.float32),
        pltpu.SemaphoreType.DMA,
    ],
    compiler_params=pltpu.CompilerParams(
        use_tc_tiling_on_sc=False, needs_layout_passes=False),
)
def _sc_gather(table_hbm, rows_hbm, cols_hbm, o_hbm,
               rows_v, cols_v, rowidx_v, par_v, g_v, obuf_v, sem):
    wid = lax.axis_index("s") * NC + lax.axis_index("c")
    base = wid * BPW
    pltpu.sync_copy(rows_hbm.at[pl.ds(base, BPW)], rows_v)
    pltpu.sync_copy(cols_hbm.at[pl.ds(base, BPW)], cols_v)
    for g in range(BPW // L):
        r = rows_v[pl.ds(g * L, L)]
        c = cols_v[pl.ds(g * L, L)]
        flat = r * SEQ + c
        rowidx_v[g // 8, pl.ds((g % 8) * L, L)] = flat >> 1
        par_v[pl.ds(g * L, L)] = (flat & 1) << 6
    copies = [
        pltpu.async_copy(
            table_hbm.at[rowidx_v.at[j]],
            g_v.at[pl.ds(j * 128, 128)], sem)
        for j in range(4)
    ]
    for cp in copies:
        cp.wait()
    iota = lax.iota(jnp.int32, L)
    for bt in range(4):
        for ch in range(8):
            k0 = bt * 128 + ch * L
            idx_r = iota + k0
            par = par_v[pl.ds(k0, L)]
            for fg in range(8):
                for fr in range(8):
                    obuf_v[fg, bt, fr, pl.ds(ch * L, L)] = plsc.load_gather(
                        g_v, [idx_r, par + (fg * 8 + fr)])
    for fg in range(8):
        pltpu.sync_copy(obuf_v.at[fg], o_hbm.at[fg, pl.ds(wid * 4, 4)])


def kernel(X, classifying_locations):
    table = jax.lax.slice(X, (0, 0, 0), (SEQ, SEQ, D)).reshape(SEQ * SEQ // 2,
                                                               2 * D)
    cl = classifying_locations.astype(jnp.int32)
    out = _sc_gather(table, cl[0], cl[1])
    return out.transpose(1, 3, 0, 2).reshape(B, D)


# speedup vs baseline: 1.3381x; 1.3381x over previous
import functools

import jax
import jax.numpy as jnp
from jax import lax
from jax.experimental import pallas as pl
from jax.experimental.pallas import tpu as pltpu
from jax.experimental.pallas import tpu_sc as plsc

B = 16384
SEQ = 200
D = 64
NC, NS, L = 2, 16, 16
NW = NC * NS
BPW = B // NW


@functools.partial(
    pl.kernel,
    out_type=jax.ShapeDtypeStruct((8, 128, 8, 128), jnp.float32),
    mesh=plsc.VectorSubcoreMesh(
        core_axis_name="c", subcore_axis_name="s",
        num_cores=NC, num_subcores=NS),
    scratch_types=[
        pltpu.VMEM((BPW,), jnp.int32),
        pltpu.VMEM((BPW,), jnp.int32),
        pltpu.VMEM((BPW,), jnp.int32),
        pltpu.VMEM((8, 4, 8, 128), jnp.int32),
        pltpu.VMEM((8, 4, 8, 128), jnp.float32),
        [pltpu.SemaphoreType.DMA] * 8,
        pltpu.SemaphoreType.DMA,
    ],
)
def _sc_gather(lflat_hbm, rows_hbm, cols_hbm, o_hbm,
               rows_v, cols_v, base_v, idx_v, gbuf_v, gsems, osem):
    wid = lax.axis_index("s") * NC + lax.axis_index("c")
    base = wid * BPW
    pltpu.sync_copy(rows_hbm.at[pl.ds(base, BPW)], rows_v)
    pltpu.sync_copy(cols_hbm.at[pl.ds(base, BPW)], cols_v)
    for g in range(BPW // L):
        r = rows_v[pl.ds(g * L, L)]
        c = cols_v[pl.ds(g * L, L)]
        base_v[pl.ds(g * L, L)] = (c << 20) | ((r >> 7) << 10) | (r & 127)
    for fg in range(8):
        for bt in range(4):
            for fr in range(8):
                off = jnp.int32((fg << 17) | (fr << 7))
                for ch in range(8):
                    idx_v[fg, bt, fr, pl.ds(ch * L, L)] = (
                        base_v[pl.ds(bt * 128 + ch * L, L)] + off)
    gathers = [
        [pltpu.async_copy(
            lflat_hbm.at[idx_v.at[fg, bt, fr]],
            gbuf_v.at[fg, bt, fr], gsems[fg])
         for bt in range(4) for fr in range(8)]
        for fg in range(8)
    ]
    outs = []
    for fg in range(8):
        for cp in gathers[fg]:
            cp.wait()
        outs.append(pltpu.async_copy(
            gbuf_v.at[fg], o_hbm.at[fg, pl.ds(wid * 4, 4)], osem))
    for cp in outs:
        cp.wait()


def kernel(X, classifying_locations):
    lflat = (X.transpose(1, 2, 0)
             .reshape(SEQ, 8, 8, 128, 128)
             .transpose(0, 1, 3, 2, 4)
             .reshape(-1))
    cl = classifying_locations.astype(jnp.int32)
    out = _sc_gather(lflat, cl[0], cl[1])
    return out.transpose(1, 3, 0, 2).reshape(B, D)


# speedup vs baseline: 1.3728x; 1.0260x over previous
import functools

import jax
import jax.numpy as jnp
from jax import lax
from jax.experimental import pallas as pl
from jax.experimental.pallas import tpu as pltpu
from jax.experimental.pallas import tpu_sc as plsc

B = 16384
SEQ = 200
D = 64
NC, NS, L = 2, 16, 16
NW = NC * NS
BPW = B // NW


@functools.partial(
    pl.kernel,
    out_type=jax.ShapeDtypeStruct((8, 128, 8, 128), jnp.float32),
    mesh=plsc.VectorSubcoreMesh(
        core_axis_name="c", subcore_axis_name="s",
        num_cores=NC, num_subcores=NS),
    scratch_types=[
        pltpu.VMEM((BPW,), jnp.int32),
        pltpu.VMEM((BPW,), jnp.int32),
        pltpu.VMEM((BPW,), jnp.int32),
        pltpu.VMEM((8, 4, 8, 128), jnp.int32),
        pltpu.VMEM((8, 4, 8, 128), jnp.float32),
        [pltpu.SemaphoreType.DMA] * 8,
        pltpu.SemaphoreType.DMA,
    ],
)
def _sc_gather(lflat_hbm, rows_hbm, cols_hbm, o_hbm,
               rows_v, cols_v, base_v, idx_v, gbuf_v, gsems, osem):
    wid = lax.axis_index("s") * NC + lax.axis_index("c")
    base = wid * BPW
    pltpu.sync_copy(rows_hbm.at[pl.ds(base, BPW)], rows_v)
    pltpu.sync_copy(cols_hbm.at[pl.ds(base, BPW)], cols_v)
    for g in range(BPW // L):
        r = rows_v[pl.ds(g * L, L)]
        c = cols_v[pl.ds(g * L, L)]
        base_v[pl.ds(g * L, L)] = (c << 20) | ((r >> 7) << 10) | (r & 127)
    gathers = []
    for fg in range(8):
        for bt in range(4):
            for fr in range(8):
                off = jnp.int32((fg << 17) | (fr << 7))
                for ch in range(8):
                    idx_v[fg, bt, fr, pl.ds(ch * L, L)] = (
                        base_v[pl.ds(bt * 128 + ch * L, L)] + off)
        gathers.append([
            pltpu.async_copy(
                lflat_hbm.at[idx_v.at[fg, bt, fr]],
                gbuf_v.at[fg, bt, fr], gsems[fg])
            for bt in range(4) for fr in range(8)
        ])
    outs = []
    for fg in range(8):
        for cp in gathers[fg]:
            cp.wait()
        outs.append(pltpu.async_copy(
            gbuf_v.at[fg], o_hbm.at[fg, pl.ds(wid * 4, 4)], osem))
    for cp in outs:
        cp.wait()


def kernel(X, classifying_locations):
    lflat = (X.transpose(1, 2, 0)
             .reshape(SEQ, 8, 8, 128, 128)
             .transpose(0, 1, 3, 2, 4)
             .reshape(-1))
    cl = classifying_locations.astype(jnp.int32)
    out = _sc_gather(lflat, cl[0], cl[1])
    return out.transpose(1, 3, 0, 2).reshape(B, D)
